# baseline (speedup 1.0000x reference)
import jax
import jax.numpy as jnp
from jax.experimental import pallas as pl
from jax.experimental.pallas import tpu as pltpu

_BATCH = 128
_SEQ = 181
_NS = 8
_FEAT = 32
_KW = 5
_ALPHA = 0.3

_SB = 16
_ROWS0 = _SEQ * _NS


def _lrelu_bf16(y):
    return jnp.maximum(y, jnp.bfloat16(_ALPHA) * y)


def _conv_folded(x, Wp, bp, F, Cin):
    SB, R, L = x.shape
    z = jnp.zeros((SB, _NS, L), x.dtype)
    xp = jnp.concatenate([z, x, z], axis=1)
    left = xp[:, 0:R, (F - 2) * Cin:]
    mid = xp[:, _NS:_NS + R, :]
    right = xp[:, 2 * _NS:2 * _NS + R, 0:2 * Cin]
    xi = jnp.concatenate([left, mid, right], axis=-1)
    d = jax.lax.dot_general(xi.reshape(SB * R, (F + 4) * Cin), Wp,
                            (((1,), (0,)), ((), ())),
                            preferred_element_type=jnp.float32)
    y = _lrelu_bf16(d.astype(jnp.bfloat16) + bp)
    return y.reshape(SB, R, Wp.shape[-1])


def _conv_lrelu(x, Wc, b):
    SB, R, Cin = x.shape
    z = jnp.zeros((SB, 2 * _NS, Cin), x.dtype)
    xp = jnp.concatenate([z, x, z], axis=1)
    cols = [xp[:, k * _NS:k * _NS + R, :] for k in range(_KW)]
    x5 = jnp.concatenate(cols, axis=-1)
    d = jax.lax.dot_general(x5.reshape(SB * R, _KW * Cin), Wc,
                            (((1,), (0,)), ((), ())),
                            preferred_element_type=jnp.float32)
    return _lrelu_bf16(d.astype(jnp.bfloat16) + b).reshape(
        SB, R, Wc.shape[-1])


def _conv_rows(x, Wr, bp, nrows):
    SB, R, L = x.shape
    z = jnp.zeros((SB, (nrows // 2) * _NS, L), x.dtype)
    xp = jnp.concatenate([z, x, z], axis=1)
    d = None
    for i in range(nrows):
        xv = xp[:, i * _NS:i * _NS + R, :].reshape(SB * R, L)
        t = jax.lax.dot_general(xv, Wr[i * L:(i + 1) * L],
                                (((1,), (0,)), ((), ())),
                                preferred_element_type=jnp.float32)
        d = t if d is None else d + t
    y = _lrelu_bf16(d.astype(jnp.bfloat16) + bp)
    return y.reshape(SB, R, Wr.shape[-1])


def _maxpool3(h, T):
    SB, R, C = h.shape
    T2 = (T // 3) * 3
    g = h[:, :T2 * _NS, :].reshape(SB, T2 // 3, 3, _NS, C)
    return g.max(axis=2).reshape(SB, (T2 // 3) * _NS, C)


def _pool_a(h):
    SB, R, L = h.shape
    g = h[:, :45 * _NS, :].reshape(SB, 15, 3, _NS, L)
    A, B, C = g[:, :, 0], g[:, :, 1], g[:, :, 2]
    t1 = jnp.concatenate([A[..., 0:64], A[..., 192:256],
                          B[..., 128:192], C[..., 64:128]], axis=-1)
    t2 = jnp.concatenate([A[..., 64:128], B[..., 0:64],
                          B[..., 192:256], C[..., 128:192]], axis=-1)
    t3 = jnp.concatenate([A[..., 128:192], B[..., 64:128],
                          C[..., 0:64], C[..., 192:256]], axis=-1)
    return jnp.maximum(t1, jnp.maximum(t2, t3)).reshape(SB, 15 * _NS, L)


def _pool_b(h):
    SB, R, L = h.shape
    g = h.reshape(SB, 10, 3, _NS, L)
    A, B, C = g[:, :, 0], g[:, :, 1], g[:, :, 2]
    t1 = jnp.concatenate([A[..., 0:128], B[..., 128:256]], axis=-1)
    t2 = jnp.concatenate([A[..., 128:256], C[..., 0:128]], axis=-1)
    t3 = jnp.concatenate([B[..., 0:128], C[..., 128:256]], axis=-1)
    return jnp.maximum(t1, jnp.maximum(t2, t3)).reshape(SB, 10 * _NS, L)


def _halve_fold(h):
    SB, R, L = h.shape
    g = h.reshape(SB, R // _NS, 1, _NS, L)
    lo = g[..., 0:L // 2]
    hi = g[..., L // 2:L]
    return jnp.concatenate([lo, hi], axis=2).reshape(SB, 2 * R, L // 2)


def _body(x_ref, W1, b1, W2, b2, W3a, b3a, W3b, b3b, W4a, b4a, W4b, b4b,
          W5, b5, Wd, bd, o_ref):
    x = x_ref[...].astype(jnp.bfloat16)
    SB = x.shape[0]
    xz = jnp.concatenate(
        [x, jnp.zeros((SB, 3 * _NS, _FEAT), x.dtype)], axis=1)
    xr = xz.reshape(SB, 46, 4, _NS, _FEAT)
    h = jnp.concatenate([xr[:, :, j] for j in range(4)],
                        axis=-1).reshape(SB, 46 * _NS, 4 * _FEAT)
    h = _conv_folded(h, W1[...], b1[...], 4, _FEAT)
    lane = jax.lax.broadcasted_iota(jnp.int32, (1, 1, 256), 2)
    tail = jnp.where(lane < 64, h[:, 45 * _NS:46 * _NS, :], jnp.bfloat16(0))
    h = jnp.concatenate([h[:, 0:45 * _NS, :], tail], axis=1)
    h = _conv_folded(h, W2[...], b2[...], 4, 64)
    h = _pool_a(h)
    h = _halve_fold(h)
    h = _conv_folded(h, W3a[...], b3a[...], 2, 64)
    h = _conv_rows(h, W3b[...], b3b[...], 3)
    h = _pool_b(h)
    h = _halve_fold(h)
    h = _conv_lrelu(h, W4a[...], b4a[...])
    h = _conv_rows(h, W4b[...], b4b[...], 5)
    h = _maxpool3(h, 20)
    h = _conv_rows(h, W5[...], b5[...], 5)
    pooled = h.astype(jnp.float32).sum(axis=1) * (1.0 / (6 * _NS))
    logits = jax.lax.dot_general(pooled, Wd[...], (((1,), (0,)), ((), ())),
                                 preferred_element_type=jnp.float32) + bd[...]
    o_ref[...] = jax.nn.sigmoid(logits)


def _pack_w_folded(W, F):
    _, Cin, Cout = W.shape
    Wb = W.astype(jnp.bfloat16)
    Wp = jnp.zeros(((F + 4) * Cin, F * Cout), jnp.bfloat16)
    for jo in range(F):
        for k in range(_KW):
            b = jo + k
            Wp = Wp.at[b * Cin:(b + 1) * Cin,
                       jo * Cout:(jo + 1) * Cout].set(Wb[k])
    return Wp


def _const_spec(shape):
    return pl.BlockSpec(shape, lambda i: (0,) * len(shape))


def kernel(sensor_features, sensor_indices, W1, b1, W2, b2, W3a, b3a,
           W3b, b3b, W4a, b4a, W4b, b4b, W5, b5, Wd, bd):
    x = sensor_features.reshape(_BATCH, _ROWS0, _FEAT)

    packed = [
        (_pack_w_folded(W1, 4), jnp.tile(b1.astype(jnp.bfloat16).reshape(1, -1), (1, 4))),
        (_pack_w_folded(W2, 4), jnp.tile(b2.astype(jnp.bfloat16).reshape(1, -1), (1, 4))),
        (_pack_w_folded(W3a, 2), jnp.tile(b3a.astype(jnp.bfloat16).reshape(1, -1), (1, 2))),
        (_pack_w_folded(W3b, 2), jnp.tile(b3b.astype(jnp.bfloat16).reshape(1, -1), (1, 2))),
    ]
    flat = [(W.astype(jnp.bfloat16).reshape(-1, W.shape[-1]),
             b.astype(jnp.bfloat16).reshape(1, -1)) for W, b in
            ((W4a, b4a), (W4b, b4b), (W5, b5))]
    bdr = bd.reshape(1, 1)

    in_specs = [pl.BlockSpec((_SB, _ROWS0, _FEAT), lambda i: (i, 0, 0))]
    ordered = []
    for W, b in packed + flat:
        in_specs.append(_const_spec(W.shape))
        in_specs.append(_const_spec(b.shape))
        ordered.extend([W, b])
    in_specs.append(_const_spec(Wd.shape))
    in_specs.append(_const_spec(bdr.shape))
    ordered.extend([Wd, bdr])

    out = pl.pallas_call(
        _body,
        grid=(_BATCH // _SB,),
        in_specs=in_specs,
        out_specs=pl.BlockSpec((_SB, 1), lambda i: (i, 0)),
        out_shape=jax.ShapeDtypeStruct((_BATCH, 1), jnp.float32),
        compiler_params=pltpu.CompilerParams(
            dimension_semantics=("parallel",)),
    )(x, *ordered)
    return out


# speedup vs baseline: 1.0236x; 1.0236x over previous
import jax
import jax.numpy as jnp
from jax.experimental import pallas as pl
from jax.experimental.pallas import tpu as pltpu

_BATCH = 128
_SEQ = 181
_NS = 8
_FEAT = 32
_KW = 5
_ALPHA = 0.3

_SB = 16
_ROWS0 = _SEQ * _NS


def _lrelu_bf16(y):
    return jnp.maximum(y, _ALPHA * y).astype(jnp.bfloat16)


def _conv_folded(x, Wp, bp, F, Cin):
    SB, R, L = x.shape
    z = jnp.zeros((SB, _NS, L), x.dtype)
    xp = jnp.concatenate([z, x, z], axis=1)
    left = xp[:, 0:R, (F - 2) * Cin:]
    mid = xp[:, _NS:_NS + R, :]
    right = xp[:, 2 * _NS:2 * _NS + R, 0:2 * Cin]
    xi = jnp.concatenate([left, mid, right], axis=-1)
    d = jax.lax.dot_general(xi.reshape(SB * R, (F + 4) * Cin), Wp,
                            (((1,), (0,)), ((), ())),
                            preferred_element_type=jnp.float32)
    y = _lrelu_bf16(d + bp)
    return y.reshape(SB, R, Wp.shape[-1])


def _conv_lrelu(x, Wc, b):
    SB, R, Cin = x.shape
    z = jnp.zeros((SB, 2 * _NS, Cin), x.dtype)
    xp = jnp.concatenate([z, x, z], axis=1)
    cols = [xp[:, k * _NS:k * _NS + R, :] for k in range(_KW)]
    x5 = jnp.concatenate(cols, axis=-1)
    d = jax.lax.dot_general(x5.reshape(SB * R, _KW * Cin), Wc,
                            (((1,), (0,)), ((), ())),
                            preferred_element_type=jnp.float32)
    return _lrelu_bf16(d + b).reshape(SB, R, Wc.shape[-1])


def _conv_rows(x, Wr, bp, nrows):
    SB, R, L = x.shape
    z = jnp.zeros((SB, (nrows // 2) * _NS, L), x.dtype)
    xp = jnp.concatenate([z, x, z], axis=1)
    d = None
    for i in range(nrows):
        xv = xp[:, i * _NS:i * _NS + R, :].reshape(SB * R, L)
        t = jax.lax.dot_general(xv, Wr[i * L:(i + 1) * L],
                                (((1,), (0,)), ((), ())),
                                preferred_element_type=jnp.float32)
        d = t if d is None else d + t
    y = _lrelu_bf16(d + bp)
    return y.reshape(SB, R, Wr.shape[-1])


def _maxpool3(h, T):
    SB, R, C = h.shape
    T2 = (T // 3) * 3
    g = h[:, :T2 * _NS, :].reshape(SB, T2 // 3, 3, _NS, C)
    return g.max(axis=2).reshape(SB, (T2 // 3) * _NS, C)


def _pool_a(h):
    SB, R, L = h.shape
    g = h[:, :45 * _NS, :].reshape(SB, 15, 3, _NS, L)
    A, B, C = g[:, :, 0], g[:, :, 1], g[:, :, 2]
    t1 = jnp.concatenate([A[..., 0:64], A[..., 192:256],
                          B[..., 128:192], C[..., 64:128]], axis=-1)
    t2 = jnp.concatenate([A[..., 64:128], B[..., 0:64],
                          B[..., 192:256], C[..., 128:192]], axis=-1)
    t3 = jnp.concatenate([A[..., 128:192], B[..., 64:128],
                          C[..., 0:64], C[..., 192:256]], axis=-1)
    return jnp.maximum(t1, jnp.maximum(t2, t3)).reshape(SB, 15 * _NS, L)


def _pool_b(h):
    SB, R, L = h.shape
    g = h.reshape(SB, 10, 3, _NS, L)
    A, B, C = g[:, :, 0], g[:, :, 1], g[:, :, 2]
    t1 = jnp.concatenate([A[..., 0:128], B[..., 128:256]], axis=-1)
    t2 = jnp.concatenate([A[..., 128:256], C[..., 0:128]], axis=-1)
    t3 = jnp.concatenate([B[..., 0:128], C[..., 128:256]], axis=-1)
    return jnp.maximum(t1, jnp.maximum(t2, t3)).reshape(SB, 10 * _NS, L)


def _halve_fold(h):
    SB, R, L = h.shape
    g = h.reshape(SB, R // _NS, 1, _NS, L)
    lo = g[..., 0:L // 2]
    hi = g[..., L // 2:L]
    return jnp.concatenate([lo, hi], axis=2).reshape(SB, 2 * R, L // 2)


def _body(x_ref, W1, b1, W2, b2, W3a, b3a, W3b, b3b, W4a, b4a, W4b, b4b,
          W5, b5, Wd, bd, o_ref):
    x = x_ref[...].astype(jnp.bfloat16)
    SB = x.shape[0]
    xz = jnp.concatenate(
        [x, jnp.zeros((SB, 3 * _NS, _FEAT), x.dtype)], axis=1)
    xr = xz.reshape(SB, 46, 4, _NS, _FEAT)
    h = jnp.concatenate([xr[:, :, j] for j in range(4)],
                        axis=-1).reshape(SB, 46 * _NS, 4 * _FEAT)
    h = _conv_folded(h, W1[...], b1[...], 4, _FEAT)
    lane = jax.lax.broadcasted_iota(jnp.int32, (1, 1, 256), 2)
    tail = jnp.where(lane < 64, h[:, 45 * _NS:46 * _NS, :], jnp.bfloat16(0))
    h = jnp.concatenate([h[:, 0:45 * _NS, :], tail], axis=1)
    h = _conv_folded(h, W2[...], b2[...], 4, 64)
    h = _pool_a(h)
    h = _halve_fold(h)
    h = _conv_folded(h, W3a[...], b3a[...], 2, 64)
    h = _conv_rows(h, W3b[...], b3b[...], 3)
    h = _pool_b(h)
    h = _halve_fold(h)
    h = _conv_lrelu(h, W4a[...], b4a[...])
    h = _conv_rows(h, W4b[...], b4b[...], 5)
    h = _maxpool3(h, 20)
    h = _conv_rows(h, W5[...], b5[...], 5)
    pooled = h.astype(jnp.float32).sum(axis=1) * (1.0 / (6 * _NS))
    logits = jax.lax.dot_general(pooled, Wd[...], (((1,), (0,)), ((), ())),
                                 preferred_element_type=jnp.float32) + bd[...]
    o_ref[...] = jax.nn.sigmoid(logits)


def _pack_w_folded(W, F):
    _, Cin, Cout = W.shape
    Wb = W.astype(jnp.bfloat16)
    Wp = jnp.zeros(((F + 4) * Cin, F * Cout), jnp.bfloat16)
    for jo in range(F):
        for k in range(_KW):
            b = jo + k
            Wp = Wp.at[b * Cin:(b + 1) * Cin,
                       jo * Cout:(jo + 1) * Cout].set(Wb[k])
    return Wp


def _const_spec(shape):
    return pl.BlockSpec(shape, lambda i: (0,) * len(shape))


def kernel(sensor_features, sensor_indices, W1, b1, W2, b2, W3a, b3a,
           W3b, b3b, W4a, b4a, W4b, b4b, W5, b5, Wd, bd):
    x = sensor_features.reshape(_BATCH, _ROWS0, _FEAT)

    packed = [
        (_pack_w_folded(W1, 4), jnp.tile(b1.reshape(1, -1), (1, 4))),
        (_pack_w_folded(W2, 4), jnp.tile(b2.reshape(1, -1), (1, 4))),
        (_pack_w_folded(W3a, 2), jnp.tile(b3a.reshape(1, -1), (1, 2))),
        (_pack_w_folded(W3b, 2), jnp.tile(b3b.reshape(1, -1), (1, 2))),
    ]
    flat = [(W.astype(jnp.bfloat16).reshape(-1, W.shape[-1]),
             b.reshape(1, -1)) for W, b in
            ((W4a, b4a), (W4b, b4b), (W5, b5))]
    bdr = bd.reshape(1, 1)

    in_specs = [pl.BlockSpec((_SB, _ROWS0, _FEAT), lambda i: (i, 0, 0))]
    ordered = []
    for W, b in packed + flat:
        in_specs.append(_const_spec(W.shape))
        in_specs.append(_const_spec(b.shape))
        ordered.extend([W, b])
    in_specs.append(_const_spec(Wd.shape))
    in_specs.append(_const_spec(bdr.shape))
    ordered.extend([Wd, bdr])

    out = pl.pallas_call(
        _body,
        grid=(_BATCH // _SB,),
        in_specs=in_specs,
        out_specs=pl.BlockSpec((_SB, 1), lambda i: (i, 0)),
        out_shape=jax.ShapeDtypeStruct((_BATCH, 1), jnp.float32),
        compiler_params=pltpu.CompilerParams(
            dimension_semantics=("parallel",)),
    )(x, *ordered)
    return out


# speedup vs baseline: 1.0606x; 1.0361x over previous
import jax
import jax.numpy as jnp
from jax.experimental import pallas as pl
from jax.experimental.pallas import tpu as pltpu

_BATCH = 128
_SEQ = 181
_NS = 8
_FEAT = 32
_KW = 5
_ALPHA = 0.3

_SB = 16
_ROWS0 = _SEQ * _NS


def _lrelu_bf16(y):
    return jnp.maximum(y, _ALPHA * y).astype(jnp.bfloat16)


def _conv_folded(x, Wp, bp, F, Cin):
    SB, R, L = x.shape
    z = jnp.zeros((SB, _NS, L), x.dtype)
    xp = jnp.concatenate([z, x, z], axis=1)
    left = xp[:, 0:R, (F - 2) * Cin:]
    mid = xp[:, _NS:_NS + R, :]
    right = xp[:, 2 * _NS:2 * _NS + R, 0:2 * Cin]
    xi = jnp.concatenate([left, mid, right], axis=-1)
    d = jax.lax.dot_general(xi.reshape(SB * R, (F + 4) * Cin), Wp,
                            (((1,), (0,)), ((), ())),
                            preferred_element_type=jnp.float32)
    y = _lrelu_bf16(d + bp)
    return y.reshape(SB, R, Wp.shape[-1])


def _conv_lrelu(x, Wc, b):
    SB, R, Cin = x.shape
    z = jnp.zeros((SB, 2 * _NS, Cin), x.dtype)
    xp = jnp.concatenate([z, x, z], axis=1)
    cols = [xp[:, k * _NS:k * _NS + R, :] for k in range(_KW)]
    x5 = jnp.concatenate(cols, axis=-1)
    d = jax.lax.dot_general(x5.reshape(SB * R, _KW * Cin), Wc,
                            (((1,), (0,)), ((), ())),
                            preferred_element_type=jnp.float32)
    return _lrelu_bf16(d + b).reshape(SB, R, Wc.shape[-1])


def _conv_rows(x, Wr, bp, nrows):
    SB, R, L = x.shape
    z = jnp.zeros((SB, (nrows // 2) * _NS, L), x.dtype)
    xp = jnp.concatenate([z, x, z], axis=1)
    d = None
    for i in range(nrows):
        xv = xp[:, i * _NS:i * _NS + R, :].reshape(SB * R, L)
        t = jax.lax.dot_general(xv, Wr[i * L:(i + 1) * L],
                                (((1,), (0,)), ((), ())),
                                preferred_element_type=jnp.float32)
        d = t if d is None else d + t
    y = _lrelu_bf16(d + bp)
    return y.reshape(SB, R, Wr.shape[-1])


def _maxpool3(h, T):
    SB, R, C = h.shape
    T2 = (T // 3) * 3
    g = h[:, :T2 * _NS, :].reshape(SB, T2 // 3, 3, _NS, C)
    return g.max(axis=2).reshape(SB, (T2 // 3) * _NS, C)


def _pool_a(h):
    SB, R, L = h.shape
    g = h[:, :45 * _NS, :].reshape(SB, 15, 3, _NS, L)
    A, B, C = g[:, :, 0], g[:, :, 1], g[:, :, 2]
    t1 = jnp.concatenate([A[..., 0:64], A[..., 192:256],
                          B[..., 128:192], C[..., 64:128]], axis=-1)
    t2 = jnp.concatenate([A[..., 64:128], B[..., 0:64],
                          B[..., 192:256], C[..., 128:192]], axis=-1)
    t3 = jnp.concatenate([A[..., 128:192], B[..., 64:128],
                          C[..., 0:64], C[..., 192:256]], axis=-1)
    return jnp.maximum(t1, jnp.maximum(t2, t3)).reshape(SB, 15 * _NS, L)


def _pool_b(h):
    SB, R, L = h.shape
    g = h.reshape(SB, 10, 3, _NS, L)
    A, B, C = g[:, :, 0], g[:, :, 1], g[:, :, 2]
    t1 = jnp.concatenate([A[..., 0:128], B[..., 128:256]], axis=-1)
    t2 = jnp.concatenate([A[..., 128:256], C[..., 0:128]], axis=-1)
    t3 = jnp.concatenate([B[..., 0:128], C[..., 128:256]], axis=-1)
    return jnp.maximum(t1, jnp.maximum(t2, t3)).reshape(SB, 10 * _NS, L)


def _halve_fold(h):
    SB, R, L = h.shape
    g = h.reshape(SB, R // _NS, 1, _NS, L)
    lo = g[..., 0:L // 2]
    hi = g[..., L // 2:L]
    return jnp.concatenate([lo, hi], axis=2).reshape(SB, 2 * R, L // 2)


def _body(x_ref, W1, b1, W2, b2, W3a, b3a, W3b, b3b, W4a, b4a, W4b, b4b,
          W5, b5, Wd, bd, o_ref):
    x = x_ref[...].astype(jnp.bfloat16)
    SB = x.shape[0]
    xz = jnp.concatenate(
        [x, jnp.zeros((SB, 3 * _NS, _FEAT), x.dtype)], axis=1)
    xr = xz.reshape(SB, 46, 4, _NS, _FEAT)
    h = jnp.concatenate([xr[:, :, j] for j in range(4)],
                        axis=-1).reshape(SB, 46 * _NS, 4 * _FEAT)
    h = _conv_folded(h, W1[...], b1[...], 4, _FEAT)
    lane = jax.lax.broadcasted_iota(jnp.int32, (1, 1, 256), 2)
    tail = jnp.where(lane < 64, h[:, 45 * _NS:46 * _NS, :], jnp.bfloat16(0))
    h = jnp.concatenate([h[:, 0:45 * _NS, :], tail], axis=1)
    h = _conv_folded(h, W2[...], b2[...], 4, 64)
    h = _pool_a(h)
    h = _halve_fold(h)
    h = _conv_folded(h, W3a[...], b3a[...], 2, 64)
    h = _conv_rows(h, W3b[...], b3b[...], 3)
    h = _pool_b(h)
    h = _halve_fold(h)
    h = _conv_lrelu(h, W4a[...], b4a[...])
    h = _conv_rows(h, W4b[...], b4b[...], 5)
    h = _maxpool3(h, 20)
    h = _conv_rows(h, W5[...], b5[...], 5)
    pooled = h.astype(jnp.float32).sum(axis=1) * (1.0 / (6 * _NS))
    logits = jax.lax.dot_general(pooled, Wd[...], (((1,), (0,)), ((), ())),
                                 preferred_element_type=jnp.float32) + bd[...]
    o_ref[...] = jax.nn.sigmoid(logits)


def _pack_w_folded(W, F):
    _, Cin, Cout = W.shape
    k = jnp.arange(F + 4)[:, None] - jnp.arange(F)[None, :]
    T = (k[:, :, None] == jnp.arange(_KW)[None, None, :]).astype(W.dtype)
    Wp = jnp.tensordot(T, W, axes=[[2], [0]])
    Wp = Wp.transpose(0, 2, 1, 3).reshape((F + 4) * Cin, F * Cout)
    return Wp.astype(jnp.bfloat16)


def _const_spec(shape):
    return pl.BlockSpec(shape, lambda i: (0,) * len(shape))


def kernel(sensor_features, sensor_indices, W1, b1, W2, b2, W3a, b3a,
           W3b, b3b, W4a, b4a, W4b, b4b, W5, b5, Wd, bd):
    x = sensor_features.reshape(_BATCH, _ROWS0, _FEAT)

    packed = [
        (_pack_w_folded(W1, 4), jnp.tile(b1.reshape(1, -1), (1, 4))),
        (_pack_w_folded(W2, 4), jnp.tile(b2.reshape(1, -1), (1, 4))),
        (_pack_w_folded(W3a, 2), jnp.tile(b3a.reshape(1, -1), (1, 2))),
        (_pack_w_folded(W3b, 2), jnp.tile(b3b.reshape(1, -1), (1, 2))),
    ]
    flat = [(W.astype(jnp.bfloat16).reshape(-1, W.shape[-1]),
             b.reshape(1, -1)) for W, b in
            ((W4a, b4a), (W4b, b4b), (W5, b5))]
    bdr = bd.reshape(1, 1)

    in_specs = [pl.BlockSpec((_SB, _ROWS0, _FEAT), lambda i: (i, 0, 0))]
    ordered = []
    for W, b in packed + flat:
        in_specs.append(_const_spec(W.shape))
        in_specs.append(_const_spec(b.shape))
        ordered.extend([W, b])
    in_specs.append(_const_spec(Wd.shape))
    in_specs.append(_const_spec(bdr.shape))
    ordered.extend([Wd, bdr])

    out = pl.pallas_call(
        _body,
        grid=(_BATCH // _SB,),
        in_specs=in_specs,
        out_specs=pl.BlockSpec((_SB, 1), lambda i: (i, 0)),
        out_shape=jax.ShapeDtypeStruct((_BATCH, 1), jnp.float32),
        compiler_params=pltpu.CompilerParams(
            dimension_semantics=("parallel",)),
    )(x, *ordered)
    return out


# speedup vs baseline: 1.0713x; 1.0101x over previous
import jax
import jax.numpy as jnp
from jax.experimental import pallas as pl
from jax.experimental.pallas import tpu as pltpu

_BATCH = 128
_SEQ = 181
_NS = 8
_FEAT = 32
_KW = 5
_ALPHA = 0.3

_SB = 16
_ROWS0 = _SEQ * _NS


def _lrelu_bf16(y):
    return jnp.maximum(y, _ALPHA * y).astype(jnp.bfloat16)


def _conv_folded(x, Wp, bp, F, Cin):
    SB, R, L = x.shape
    z = jnp.zeros((SB, _NS, L), x.dtype)
    xp = jnp.concatenate([z, x, z], axis=1)
    left = xp[:, 0:R, (F - 2) * Cin:]
    mid = xp[:, _NS:_NS + R, :]
    right = xp[:, 2 * _NS:2 * _NS + R, 0:2 * Cin]
    xi = jnp.concatenate([left, mid, right], axis=-1)
    d = jax.lax.dot_general(xi.reshape(SB * R, (F + 4) * Cin), Wp,
                            (((1,), (0,)), ((), ())),
                            preferred_element_type=jnp.float32)
    y = _lrelu_bf16(d + bp)
    return y.reshape(SB, R, Wp.shape[-1])


def _conv_lrelu(x, Wc, b):
    SB, R, Cin = x.shape
    Wc = Wc.astype(jnp.bfloat16)
    z = jnp.zeros((SB, 2 * _NS, Cin), x.dtype)
    xp = jnp.concatenate([z, x, z], axis=1)
    cols = [xp[:, k * _NS:k * _NS + R, :] for k in range(_KW)]
    x5 = jnp.concatenate(cols, axis=-1)
    d = jax.lax.dot_general(x5.reshape(SB * R, _KW * Cin), Wc,
                            (((1,), (0,)), ((), ())),
                            preferred_element_type=jnp.float32)
    return _lrelu_bf16(d + b).reshape(SB, R, Wc.shape[-1])


def _conv_rows(x, Wr, bp, nrows):
    SB, R, L = x.shape
    Wr = Wr.astype(jnp.bfloat16)
    z = jnp.zeros((SB, (nrows // 2) * _NS, L), x.dtype)
    xp = jnp.concatenate([z, x, z], axis=1)
    d = None
    for i in range(nrows):
        xv = xp[:, i * _NS:i * _NS + R, :].reshape(SB * R, L)
        t = jax.lax.dot_general(xv, Wr[i * L:(i + 1) * L],
                                (((1,), (0,)), ((), ())),
                                preferred_element_type=jnp.float32)
        d = t if d is None else d + t
    y = _lrelu_bf16(d + bp)
    return y.reshape(SB, R, Wr.shape[-1])


def _maxpool3(h, T):
    SB, R, C = h.shape
    T2 = (T // 3) * 3
    g = h[:, :T2 * _NS, :].reshape(SB, T2 // 3, 3, _NS, C)
    return g.max(axis=2).reshape(SB, (T2 // 3) * _NS, C)


def _pool_a(h):
    SB, R, L = h.shape
    g = h[:, :45 * _NS, :].reshape(SB, 15, 3, _NS, L)
    A, B, C = g[:, :, 0], g[:, :, 1], g[:, :, 2]
    t1 = jnp.concatenate([A[..., 0:64], A[..., 192:256],
                          B[..., 128:192], C[..., 64:128]], axis=-1)
    t2 = jnp.concatenate([A[..., 64:128], B[..., 0:64],
                          B[..., 192:256], C[..., 128:192]], axis=-1)
    t3 = jnp.concatenate([A[..., 128:192], B[..., 64:128],
                          C[..., 0:64], C[..., 192:256]], axis=-1)
    return jnp.maximum(t1, jnp.maximum(t2, t3)).reshape(SB, 15 * _NS, L)


def _pool_b(h):
    SB, R, L = h.shape
    g = h.reshape(SB, 10, 3, _NS, L)
    A, B, C = g[:, :, 0], g[:, :, 1], g[:, :, 2]
    t1 = jnp.concatenate([A[..., 0:128], B[..., 128:256]], axis=-1)
    t2 = jnp.concatenate([A[..., 128:256], C[..., 0:128]], axis=-1)
    t3 = jnp.concatenate([B[..., 0:128], C[..., 128:256]], axis=-1)
    return jnp.maximum(t1, jnp.maximum(t2, t3)).reshape(SB, 10 * _NS, L)


def _halve_fold(h):
    SB, R, L = h.shape
    g = h.reshape(SB, R // _NS, 1, _NS, L)
    lo = g[..., 0:L // 2]
    hi = g[..., L // 2:L]
    return jnp.concatenate([lo, hi], axis=2).reshape(SB, 2 * R, L // 2)


def _body(x_ref, W1, b1, W2, b2, W3a, b3a, W3b, b3b, W4a, b4a, W4b, b4b,
          W5, b5, Wd, bd, o_ref):
    x = x_ref[...].astype(jnp.bfloat16)
    SB = x.shape[0]
    xz = jnp.concatenate(
        [x, jnp.zeros((SB, 3 * _NS, _FEAT), x.dtype)], axis=1)
    xr = xz.reshape(SB, 46, 4, _NS, _FEAT)
    h = jnp.concatenate([xr[:, :, j] for j in range(4)],
                        axis=-1).reshape(SB, 46 * _NS, 4 * _FEAT)
    h = _conv_folded(h, W1[...], b1[...], 4, _FEAT)
    lane = jax.lax.broadcasted_iota(jnp.int32, (1, 1, 256), 2)
    tail = jnp.where(lane < 64, h[:, 45 * _NS:46 * _NS, :], jnp.bfloat16(0))
    h = jnp.concatenate([h[:, 0:45 * _NS, :], tail], axis=1)
    h = _conv_folded(h, W2[...], b2[...], 4, 64)
    h = _pool_a(h)
    h = _halve_fold(h)
    h = _conv_folded(h, W3a[...], b3a[...], 2, 64)
    h = _conv_rows(h, W3b[...], b3b[...], 3)
    h = _pool_b(h)
    h = _halve_fold(h)
    h = _conv_lrelu(h, W4a[...], b4a[...])
    h = _conv_rows(h, W4b[...], b4b[...], 5)
    h = _maxpool3(h, 20)
    h = _conv_rows(h, W5[...], b5[...], 5)
    pooled = h.astype(jnp.float32).sum(axis=1) * (1.0 / (6 * _NS))
    logits = jax.lax.dot_general(pooled, Wd[...], (((1,), (0,)), ((), ())),
                                 preferred_element_type=jnp.float32) + bd[...]
    o_ref[...] = jax.nn.sigmoid(logits)


def _pack_w_folded(W, F):
    _, Cin, Cout = W.shape
    k = jnp.arange(F + 4)[:, None] - jnp.arange(F)[None, :]
    T = (k[:, :, None] == jnp.arange(_KW)[None, None, :]).astype(W.dtype)
    Wp = jnp.tensordot(T, W, axes=[[2], [0]])
    Wp = Wp.transpose(0, 2, 1, 3).reshape((F + 4) * Cin, F * Cout)
    return Wp.astype(jnp.bfloat16)


def _const_spec(shape):
    return pl.BlockSpec(shape, lambda i: (0,) * len(shape))


def kernel(sensor_features, sensor_indices, W1, b1, W2, b2, W3a, b3a,
           W3b, b3b, W4a, b4a, W4b, b4b, W5, b5, Wd, bd):
    x = sensor_features.reshape(_BATCH, _ROWS0, _FEAT)

    packed = [
        (_pack_w_folded(W1, 4), jnp.tile(b1.reshape(1, -1), (1, 4))),
        (_pack_w_folded(W2, 4), jnp.tile(b2.reshape(1, -1), (1, 4))),
        (_pack_w_folded(W3a, 2), jnp.tile(b3a.reshape(1, -1), (1, 2))),
        (_pack_w_folded(W3b, 2), jnp.tile(b3b.reshape(1, -1), (1, 2))),
    ]
    flat = [(W.reshape(-1, W.shape[-1]),
             b.reshape(1, -1)) for W, b in
            ((W4a, b4a), (W4b, b4b), (W5, b5))]
    bdr = bd.reshape(1, 1)

    in_specs = [pl.BlockSpec((_SB, _ROWS0, _FEAT), lambda i: (i, 0, 0))]
    ordered = []
    for W, b in packed + flat:
        in_specs.append(_const_spec(W.shape))
        in_specs.append(_const_spec(b.shape))
        ordered.extend([W, b])
    in_specs.append(_const_spec(Wd.shape))
    in_specs.append(_const_spec(bdr.shape))
    ordered.extend([Wd, bdr])

    out = pl.pallas_call(
        _body,
        grid=(_BATCH // _SB,),
        in_specs=in_specs,
        out_specs=pl.BlockSpec((_SB, 1), lambda i: (i, 0)),
        out_shape=jax.ShapeDtypeStruct((_BATCH, 1), jnp.float32),
        compiler_params=pltpu.CompilerParams(
            dimension_semantics=("parallel",)),
    )(x, *ordered)
    return out
